# baseline (speedup 1.0000x reference)
import jax
import jax.numpy as jnp
from jax.experimental import pallas as pl

_N = 32768
_K = 64
_BLK = 8


def _topk_block(x_ref, vals_ref, idx_ref):
    x = x_ref[...]
    col = jax.lax.broadcasted_iota(jnp.int32, (_BLK, _N), 1)
    lane = jax.lax.broadcasted_iota(jnp.int32, (_BLK, _K), 1)
    neg = jnp.float32(-jnp.inf)

    def body(k, carry):
        x, vals, idxs = carry
        m = jnp.max(x, axis=1, keepdims=True)
        i = jnp.min(jnp.where(x == m, col, _N), axis=1, keepdims=True)
        x = jnp.where(col == i, neg, x)
        vals = jnp.where(lane == k, m, vals)
        idxs = jnp.where(lane == k, i, idxs)
        return x, vals, idxs

    _, vals, idxs = jax.lax.fori_loop(
        0, _K, body,
        (x, jnp.zeros((_BLK, _K), jnp.float32), jnp.zeros((_BLK, _K), jnp.int32)),
    )
    vals_ref[...] = vals
    idx_ref[...] = idxs


def kernel(accumulated):
    vals, idx = pl.pallas_call(
        _topk_block,
        grid=(accumulated.shape[0] // _BLK,),
        in_specs=[pl.BlockSpec((_BLK, _N), lambda i: (i, 0))],
        out_specs=[
            pl.BlockSpec((_BLK, _K), lambda i: (i, 0)),
            pl.BlockSpec((_BLK, _K), lambda i: (i, 0)),
        ],
        out_shape=[
            jax.ShapeDtypeStruct((accumulated.shape[0], _K), jnp.float32),
            jax.ShapeDtypeStruct((accumulated.shape[0], _K), jnp.int32),
        ],
    )(accumulated)
    return vals, idx.astype(jnp.int64), accumulated


# speedup vs baseline: 7.2229x; 7.2229x over previous
import functools

import jax
import jax.numpy as jnp
from jax import lax
from jax.experimental import pallas as pl
from jax.experimental.pallas import tpu as pltpu
from jax.experimental.pallas import tpu_sc as plsc

_N = 32768
_K = 64
_ROWS = 128
_NW = 32
_RPW = _ROWS // _NW
_CAP = _N + 256

_i32 = jnp.int32
_f32 = jnp.float32


def _vsd(x):
    i = lax.iota(_i32, 16)
    for k in (2, 4, 8, 16):
        j = k // 2
        while j >= 1:
            perm = i ^ j
            minm = ((i & k) == 0) == ((i & j) == 0)
            xp = _take16(x, perm)
            x = jnp.where(minm, jnp.minimum(x, xp), jnp.maximum(x, xp))
            j //= 2
    return lax.rev(x, (0,))


def _rev(x):
    return lax.rev(x, (0,))


def _clean32(x0, x1):
    hi = jnp.maximum(x0, x1)
    lo = jnp.minimum(x0, x1)
    return _vsd(hi), _vsd(lo)


def _merge2(a, b):
    rb = _rev(b)
    return _vsd(jnp.maximum(a, rb)), _vsd(jnp.minimum(a, rb))


def _merge4(a0, a1, b0, b1):
    r0, r1 = _rev(b1), _rev(b0)
    h0, h1 = jnp.maximum(a0, r0), jnp.maximum(a1, r1)
    l0, l1 = jnp.minimum(a0, r0), jnp.minimum(a1, r1)
    ch = _clean32(h0, h1)
    cl = _clean32(l0, l1)
    return ch[0], ch[1], cl[0], cl[1]


def _sort64(v0, v1, v2, v3):
    a = _merge2(_vsd(v0), _vsd(v1))
    b = _merge2(_vsd(v2), _vsd(v3))
    return _merge4(a[0], a[1], b[0], b[1])


def _mtop64(r, c):
    rc = (_rev(c[3]), _rev(c[2]), _rev(c[1]), _rev(c[0]))
    t0 = jnp.maximum(r[0], rc[0])
    t1 = jnp.maximum(r[1], rc[1])
    t2 = jnp.maximum(r[2], rc[2])
    t3 = jnp.maximum(r[3], rc[3])
    p0, p1 = jnp.maximum(t0, t2), jnp.maximum(t1, t3)
    q0, q1 = jnp.minimum(t0, t2), jnp.minimum(t1, t3)
    ch = _clean32(p0, p1)
    cl = _clean32(q0, q1)
    return ch[0], ch[1], cl[0], cl[1]


def _take16(v, idx):
    dnums = lax.GatherDimensionNumbers(
        offset_dims=(), collapsed_slice_dims=(0,), start_index_map=(0,))
    return lax.gather(v, idx[:, None], dnums, (1,),
                      mode=lax.GatherScatterMode.PROMISE_IN_BOUNDS)


def _splat_i32(s):
    return jnp.full((16,), s, _i32)


def _splat_f32(s):
    return jnp.full((16,), s, _f32)


def _prefix_total(m):
    i = lax.iota(_i32, 16)
    p = jnp.where(m, 1, 0).astype(_i32)
    for d in (1, 2, 4, 8):
        shifted = _take16(p, jnp.maximum(i - d, 0))
        p = p + jnp.where(i >= d, shifted, 0)
    total = _take16(p, jnp.full((16,), 15, _i32))
    return p, total


def _any_splat(m):
    i = lax.iota(_i32, 16)
    s = jnp.where(m, 1, 0).astype(_i32)
    for d in (1, 2, 4, 8):
        s = jnp.maximum(s, _take16(s, i ^ d))
    return s > 0


def _sc_body(acc_hbm, outv_hbm, outi_hbm,
             row_v, candv, candi, hitids, selv, seli, outv, outi):
    wid = lax.axis_index("c") * 16 + lax.axis_index("s")
    iota = lax.iota(_i32, 16)
    lane0 = iota == 0
    neginf = _splat_f32(-jnp.inf)
    zeros32 = _splat_i32(0)
    one = _splat_i32(1)

    def do_row(rr, _):
        row_id = wid * _RPW + rr
        pltpu.sync_copy(acc_hbm.at[row_id], row_v)

        def p1_body(it, m):
            base = it * 256
            return tuple(
                jnp.maximum(m[j], row_v[pl.ds(base + j * 16, 16)])
                for j in range(16))

        m = lax.fori_loop(0, 128, p1_body, (neginf,) * 16)
        a = _sort64(m[0], m[1], m[2], m[3])
        b = _sort64(m[4], m[5], m[6], m[7])
        c4 = _sort64(m[8], m[9], m[10], m[11])
        d = _sort64(m[12], m[13], m[14], m[15])
        t = _mtop64(_mtop64(a, b), _mtop64(c4, d))
        tau = _splat_f32(t[3][15])

        def p2a_body(blk, hc):
            base = blk * 64
            v0 = row_v[pl.ds(base, 16)]
            v1 = row_v[pl.ds(base + 16, 16)]
            v2 = row_v[pl.ds(base + 32, 16)]
            v3 = row_v[pl.ds(base + 48, 16)]
            bm = jnp.maximum(jnp.maximum(v0, v1), jnp.maximum(v2, v3))
            hit = _any_splat(bm >= tau)
            plsc.store_scatter(hitids, [hc], _splat_i32(blk),
                               mask=hit & lane0)
            return hc + jnp.where(hit, one, zeros32)

        hc = lax.fori_loop(0, _N // 64, p2a_body, zeros32)
        nhit = hc[0]

        def p2b_body(h, cc):
            hv = hitids[pl.ds((h // 16) * 16, 16)]
            blk = _take16(hv, jnp.full((16,), h % 16, _i32))[0]
            base = blk * 64
            for k in range(4):
                v = row_v[pl.ds(base + k * 16, 16)]
                msk = v >= tau
                pre, tot = _prefix_total(msk)
                pos = cc + pre - 1
                plsc.store_scatter(candv, [pos], v, mask=msk)
                gi = _splat_i32(base + k * 16) + iota
                plsc.store_scatter(candi, [pos], gi, mask=msk)
                cc = cc + tot
            return cc

        cc = lax.fori_loop(0, nhit, p2b_body, zeros32)
        nc = cc[0]

        for k in range(4):
            pidx = cc + (iota + 16 * k)
            plsc.store_scatter(candv, [pidx], neginf)
            plsc.store_scatter(candi, [pidx], _splat_i32(2 ** 30))

        r0 = _sort64(candv[pl.ds(0, 16)], candv[pl.ds(16, 16)],
                     candv[pl.ds(32, 16)], candv[pl.ds(48, 16)])
        nchunks = (nc + 63) // 64

        def p3_body(c5, r):
            bb = c5 * 64
            cs = _sort64(candv[pl.ds(bb, 16)], candv[pl.ds(bb + 16, 16)],
                         candv[pl.ds(bb + 32, 16)], candv[pl.ds(bb + 48, 16)])
            return _mtop64(r, cs)

        r = lax.fori_loop(1, nchunks, p3_body, r0)
        v64 = _splat_f32(r[3][15])

        nv = (nc + 15) // 16

        def p4gt_body(i, sc):
            v = candv[pl.ds(i * 16, 16)]
            ii = candi[pl.ds(i * 16, 16)]
            msk = v > v64
            pre, tot = _prefix_total(msk)
            pos = sc + pre - 1
            plsc.store_scatter(selv, [pos], v, mask=msk)
            plsc.store_scatter(seli, [pos], ii, mask=msk)
            return sc + tot

        sc = lax.fori_loop(0, nv, p4gt_body, zeros32)

        def p4eq_body(i, sc):
            v = candv[pl.ds(i * 16, 16)]
            ii = candi[pl.ds(i * 16, 16)]
            msk0 = v == v64
            pre, _ = _prefix_total(msk0)
            pos = sc + pre - 1
            msk = msk0 & (pos < _K)
            _, tot = _prefix_total(msk)
            plsc.store_scatter(selv, [pos], v, mask=msk)
            plsc.store_scatter(seli, [pos], ii, mask=msk)
            return sc + tot

        lax.fori_loop(0, nv, p4eq_body, sc)

        sv = tuple(selv[pl.ds(k * 16, 16)] for k in range(4))
        si = tuple(seli[pl.ds(k * 16, 16)] for k in range(4))

        def p5_body(j, ranks):
            jb = (j // 16) * 16
            jl = jnp.full((16,), j % 16, _i32)
            bv = _take16(selv[pl.ds(jb, 16)], jl)
            bi = _take16(seli[pl.ds(jb, 16)], jl)
            out = []
            for k in range(4):
                beats = (bv > sv[k]) | ((bv == sv[k]) & (bi < si[k]))
                out.append(ranks[k] + jnp.where(beats, one, zeros32))
            return tuple(out)

        ranks = lax.fori_loop(0, _K, p5_body, (zeros32,) * 4)
        rsp = _splat_i32(rr)
        for k in range(4):
            plsc.store_scatter(outv, [rsp, ranks[k]], sv[k])
            plsc.store_scatter(outi, [rsp, ranks[k]], si[k])
        return 0

    lax.fori_loop(0, _RPW, do_row, 0)
    pltpu.sync_copy(outv, outv_hbm.at[pl.ds(wid * _RPW, _RPW)])
    pltpu.sync_copy(outi, outi_hbm.at[pl.ds(wid * _RPW, _RPW)])


_sc_topk = pl.kernel(
    _sc_body,
    out_type=[
        jax.ShapeDtypeStruct((_ROWS, _K), _f32),
        jax.ShapeDtypeStruct((_ROWS, _K), _i32),
    ],
    mesh=plsc.VectorSubcoreMesh(core_axis_name="c", subcore_axis_name="s"),
    compiler_params=pltpu.CompilerParams(needs_layout_passes=False),
    scratch_types=[
        pltpu.VMEM((_N,), _f32),
        pltpu.VMEM((_CAP,), _f32),
        pltpu.VMEM((_CAP,), _i32),
        pltpu.VMEM((_N // 64,), _i32),
        pltpu.VMEM((80,), _f32),
        pltpu.VMEM((80,), _i32),
        pltpu.VMEM((_RPW, _K), _f32),
        pltpu.VMEM((_RPW, _K), _i32),
    ],
)


def kernel(accumulated):
    vals, idx = _sc_topk(accumulated)
    return vals, idx.astype(jnp.int64), accumulated


# speedup vs baseline: 9.3756x; 1.2980x over previous
import functools

import jax
import jax.numpy as jnp
from jax import lax
from jax.experimental import pallas as pl
from jax.experimental.pallas import tpu as pltpu
from jax.experimental.pallas import tpu_sc as plsc

_N = 32768
_K = 64
_ROWS = 128
_NW = 32
_RPW = _ROWS // _NW
_CAP = _N + 256

_i32 = jnp.int32
_f32 = jnp.float32


def _vsd(x):
    return lax.rev(lax.sort(x), (0,))


def _rev(x):
    return lax.rev(x, (0,))


def _clean32(x0, x1):
    hi = jnp.maximum(x0, x1)
    lo = jnp.minimum(x0, x1)
    return _vsd(hi), _vsd(lo)


def _merge2(a, b):
    rb = _rev(b)
    return _vsd(jnp.maximum(a, rb)), _vsd(jnp.minimum(a, rb))


def _merge4(a0, a1, b0, b1):
    r0, r1 = _rev(b1), _rev(b0)
    h0, h1 = jnp.maximum(a0, r0), jnp.maximum(a1, r1)
    l0, l1 = jnp.minimum(a0, r0), jnp.minimum(a1, r1)
    ch = _clean32(h0, h1)
    cl = _clean32(l0, l1)
    return ch[0], ch[1], cl[0], cl[1]


def _sort64(v0, v1, v2, v3):
    a = _merge2(_vsd(v0), _vsd(v1))
    b = _merge2(_vsd(v2), _vsd(v3))
    return _merge4(a[0], a[1], b[0], b[1])


def _mtop64(r, c):
    rc = (_rev(c[3]), _rev(c[2]), _rev(c[1]), _rev(c[0]))
    t0 = jnp.maximum(r[0], rc[0])
    t1 = jnp.maximum(r[1], rc[1])
    t2 = jnp.maximum(r[2], rc[2])
    t3 = jnp.maximum(r[3], rc[3])
    p0, p1 = jnp.maximum(t0, t2), jnp.maximum(t1, t3)
    q0, q1 = jnp.minimum(t0, t2), jnp.minimum(t1, t3)
    ch = _clean32(p0, p1)
    cl = _clean32(q0, q1)
    return ch[0], ch[1], cl[0], cl[1]


def _take16(v, idx):
    dnums = lax.GatherDimensionNumbers(
        offset_dims=(), collapsed_slice_dims=(0,), start_index_map=(0,))
    return lax.gather(v, idx[:, None], dnums, (1,),
                      mode=lax.GatherScatterMode.PROMISE_IN_BOUNDS)


def _splat_i32(s):
    return jnp.full((16,), s, _i32)


def _splat_f32(s):
    return jnp.full((16,), s, _f32)


def _prefix_total(m):
    p = plsc.cumsum(jnp.where(m, 1, 0).astype(_i32))
    total = _take16(p, jnp.full((16,), 15, _i32))
    return p, total


def _any_splat(m):
    return plsc.all_reduce_population_count(m) > 0


def _sc_body(acc_hbm, outv_hbm, outi_hbm,
             row_v, candv, candi, hitids, selv, seli, outv, outi):
    wid = lax.axis_index("c") * 16 + lax.axis_index("s")
    iota = lax.iota(_i32, 16)
    lane0 = iota == 0
    neginf = _splat_f32(-jnp.inf)
    zeros32 = _splat_i32(0)
    one = _splat_i32(1)

    def do_row(rr, _):
        row_id = wid * _RPW + rr
        pltpu.sync_copy(acc_hbm.at[row_id], row_v)

        def p1_body(it, m):
            base = it * 256
            return tuple(
                jnp.maximum(m[j], row_v[pl.ds(base + j * 16, 16)])
                for j in range(16))

        m = lax.fori_loop(0, 128, p1_body, (neginf,) * 16)
        a = _sort64(m[0], m[1], m[2], m[3])
        b = _sort64(m[4], m[5], m[6], m[7])
        c4 = _sort64(m[8], m[9], m[10], m[11])
        d = _sort64(m[12], m[13], m[14], m[15])
        t = _mtop64(_mtop64(a, b), _mtop64(c4, d))
        tau = _splat_f32(t[3][15])

        def p2a_body(blk, hc):
            base = blk * 64
            v0 = row_v[pl.ds(base, 16)]
            v1 = row_v[pl.ds(base + 16, 16)]
            v2 = row_v[pl.ds(base + 32, 16)]
            v3 = row_v[pl.ds(base + 48, 16)]
            bm = jnp.maximum(jnp.maximum(v0, v1), jnp.maximum(v2, v3))
            hit = _any_splat(bm >= tau)
            plsc.store_scatter(hitids, [hc], _splat_i32(blk),
                               mask=hit & lane0)
            return hc + jnp.where(hit, one, zeros32)

        hc = lax.fori_loop(0, _N // 64, p2a_body, zeros32)
        nhit = hc[0]

        def p2b_body(h, cc):
            hv = hitids[pl.ds((h // 16) * 16, 16)]
            blk = _take16(hv, jnp.full((16,), h % 16, _i32))[0]
            base = blk * 64
            for k in range(4):
                v = row_v[pl.ds(base + k * 16, 16)]
                msk = v >= tau
                pre, tot = _prefix_total(msk)
                pos = cc + pre - 1
                plsc.store_scatter(candv, [pos], v, mask=msk)
                gi = _splat_i32(base + k * 16) + iota
                plsc.store_scatter(candi, [pos], gi, mask=msk)
                cc = cc + tot
            return cc

        cc = lax.fori_loop(0, nhit, p2b_body, zeros32)
        nc = cc[0]

        for k in range(4):
            pidx = cc + (iota + 16 * k)
            plsc.store_scatter(candv, [pidx], neginf)
            plsc.store_scatter(candi, [pidx], _splat_i32(2 ** 30))

        r0 = _sort64(candv[pl.ds(0, 16)], candv[pl.ds(16, 16)],
                     candv[pl.ds(32, 16)], candv[pl.ds(48, 16)])
        nchunks = (nc + 63) // 64

        def p3_body(c5, r):
            bb = c5 * 64
            cs = _sort64(candv[pl.ds(bb, 16)], candv[pl.ds(bb + 16, 16)],
                         candv[pl.ds(bb + 32, 16)], candv[pl.ds(bb + 48, 16)])
            return _mtop64(r, cs)

        r = lax.fori_loop(1, nchunks, p3_body, r0)
        v64 = _splat_f32(r[3][15])

        nv = (nc + 15) // 16

        def p4gt_body(i, sc):
            v = candv[pl.ds(i * 16, 16)]
            ii = candi[pl.ds(i * 16, 16)]
            msk = v > v64
            pre, tot = _prefix_total(msk)
            pos = sc + pre - 1
            plsc.store_scatter(selv, [pos], v, mask=msk)
            plsc.store_scatter(seli, [pos], ii, mask=msk)
            return sc + tot

        sc = lax.fori_loop(0, nv, p4gt_body, zeros32)

        def p4eq_body(i, sc):
            v = candv[pl.ds(i * 16, 16)]
            ii = candi[pl.ds(i * 16, 16)]
            msk0 = v == v64
            pre, _ = _prefix_total(msk0)
            pos = sc + pre - 1
            msk = msk0 & (pos < _K)
            _, tot = _prefix_total(msk)
            plsc.store_scatter(selv, [pos], v, mask=msk)
            plsc.store_scatter(seli, [pos], ii, mask=msk)
            return sc + tot

        lax.fori_loop(0, nv, p4eq_body, sc)

        sv = tuple(selv[pl.ds(k * 16, 16)] for k in range(4))
        si = tuple(seli[pl.ds(k * 16, 16)] for k in range(4))

        def p5_body(j, ranks):
            jb = (j // 16) * 16
            jl = jnp.full((16,), j % 16, _i32)
            bv = _take16(selv[pl.ds(jb, 16)], jl)
            bi = _take16(seli[pl.ds(jb, 16)], jl)
            out = []
            for k in range(4):
                beats = (bv > sv[k]) | ((bv == sv[k]) & (bi < si[k]))
                out.append(ranks[k] + jnp.where(beats, one, zeros32))
            return tuple(out)

        ranks = lax.fori_loop(0, _K, p5_body, (zeros32,) * 4)
        rsp = _splat_i32(rr)
        for k in range(4):
            plsc.store_scatter(outv, [rsp, ranks[k]], sv[k])
            plsc.store_scatter(outi, [rsp, ranks[k]], si[k])
        return 0

    lax.fori_loop(0, _RPW, do_row, 0)
    pltpu.sync_copy(outv, outv_hbm.at[pl.ds(wid * _RPW, _RPW)])
    pltpu.sync_copy(outi, outi_hbm.at[pl.ds(wid * _RPW, _RPW)])


_sc_topk = pl.kernel(
    _sc_body,
    out_type=[
        jax.ShapeDtypeStruct((_ROWS, _K), _f32),
        jax.ShapeDtypeStruct((_ROWS, _K), _i32),
    ],
    mesh=plsc.VectorSubcoreMesh(core_axis_name="c", subcore_axis_name="s"),
    compiler_params=pltpu.CompilerParams(needs_layout_passes=False),
    scratch_types=[
        pltpu.VMEM((_N,), _f32),
        pltpu.VMEM((_CAP,), _f32),
        pltpu.VMEM((_CAP,), _i32),
        pltpu.VMEM((_N // 64,), _i32),
        pltpu.VMEM((80,), _f32),
        pltpu.VMEM((80,), _i32),
        pltpu.VMEM((_RPW, _K), _f32),
        pltpu.VMEM((_RPW, _K), _i32),
    ],
)


def kernel(accumulated):
    vals, idx = _sc_topk(accumulated)
    return vals, idx.astype(jnp.int64), accumulated


# speedup vs baseline: 9.5352x; 1.0170x over previous
import functools

import jax
import jax.numpy as jnp
from jax import lax
from jax.experimental import pallas as pl
from jax.experimental.pallas import tpu as pltpu
from jax.experimental.pallas import tpu_sc as plsc

_N = 32768
_K = 64
_ROWS = 128
_NW = 32
_RPW = _ROWS // _NW
_CAP = _N + 256

_i32 = jnp.int32
_f32 = jnp.float32


def _vsd(x):
    return lax.rev(lax.sort(x), (0,))


def _rev(x):
    return lax.rev(x, (0,))


def _clean32(x0, x1):
    hi = jnp.maximum(x0, x1)
    lo = jnp.minimum(x0, x1)
    return _vsd(hi), _vsd(lo)


def _merge2(a, b):
    rb = _rev(b)
    return _vsd(jnp.maximum(a, rb)), _vsd(jnp.minimum(a, rb))


def _merge4(a0, a1, b0, b1):
    r0, r1 = _rev(b1), _rev(b0)
    h0, h1 = jnp.maximum(a0, r0), jnp.maximum(a1, r1)
    l0, l1 = jnp.minimum(a0, r0), jnp.minimum(a1, r1)
    ch = _clean32(h0, h1)
    cl = _clean32(l0, l1)
    return ch[0], ch[1], cl[0], cl[1]


def _sort64(v0, v1, v2, v3):
    a = _merge2(_vsd(v0), _vsd(v1))
    b = _merge2(_vsd(v2), _vsd(v3))
    return _merge4(a[0], a[1], b[0], b[1])


def _mtop64(r, c):
    rc = (_rev(c[3]), _rev(c[2]), _rev(c[1]), _rev(c[0]))
    t0 = jnp.maximum(r[0], rc[0])
    t1 = jnp.maximum(r[1], rc[1])
    t2 = jnp.maximum(r[2], rc[2])
    t3 = jnp.maximum(r[3], rc[3])
    p0, p1 = jnp.maximum(t0, t2), jnp.maximum(t1, t3)
    q0, q1 = jnp.minimum(t0, t2), jnp.minimum(t1, t3)
    ch = _clean32(p0, p1)
    cl = _clean32(q0, q1)
    return ch[0], ch[1], cl[0], cl[1]


def _take16(v, idx):
    dnums = lax.GatherDimensionNumbers(
        offset_dims=(), collapsed_slice_dims=(0,), start_index_map=(0,))
    return lax.gather(v, idx[:, None], dnums, (1,),
                      mode=lax.GatherScatterMode.PROMISE_IN_BOUNDS)


def _splat_i32(s):
    return jnp.full((16,), s, _i32)


def _splat_f32(s):
    return jnp.full((16,), s, _f32)


def _prefix_total(m):
    p = plsc.cumsum(jnp.where(m, 1, 0).astype(_i32))
    total = _take16(p, jnp.full((16,), 15, _i32))
    return p, total


def _any_splat(m):
    return plsc.all_reduce_population_count(m) > 0


def _sc_body(acc_hbm, outv_hbm, outi_hbm,
             row_v, candv, candi, hitids, selv, seli, outv, outi,
             blockred, superred, superids):
    wid = lax.axis_index("c") * 16 + lax.axis_index("s")
    iota = lax.iota(_i32, 16)
    lane0 = iota == 0
    neginf = _splat_f32(-jnp.inf)
    zeros32 = _splat_i32(0)
    one = _splat_i32(1)

    def do_row(rr, _):
        row_id = wid * _RPW + rr
        pltpu.sync_copy(acc_hbm.at[row_id], row_v)

        def p1_body(it4, m):
            m = list(m)
            for u in range(4):
                sup = it4 * 4 + u
                base = sup * 256
                sred = None
                for q in range(4):
                    blk = sup * 4 + q
                    v0 = row_v[pl.ds(base + q * 64, 16)]
                    v1 = row_v[pl.ds(base + q * 64 + 16, 16)]
                    v2 = row_v[pl.ds(base + q * 64 + 32, 16)]
                    v3 = row_v[pl.ds(base + q * 64 + 48, 16)]
                    r = jnp.maximum(jnp.maximum(v0, v1), jnp.maximum(v2, v3))
                    blockred[pl.ds(blk * 16, 16)] = r
                    m[u * 4 + q] = jnp.maximum(m[u * 4 + q], r)
                    sred = r if sred is None else jnp.maximum(sred, r)
                superred[pl.ds(sup * 16, 16)] = sred
            return tuple(m)

        m = lax.fori_loop(0, 32, p1_body, (neginf,) * 16)
        a = _sort64(m[0], m[1], m[2], m[3])
        b = _sort64(m[4], m[5], m[6], m[7])
        c4 = _sort64(m[8], m[9], m[10], m[11])
        d = _sort64(m[12], m[13], m[14], m[15])
        t = _mtop64(_mtop64(a, b), _mtop64(c4, d))
        tau = _splat_f32(t[3][15])

        def p2s_body(s, sc_):
            srv = superred[pl.ds(s * 16, 16)]
            hit = _any_splat(srv >= tau)
            plsc.store_scatter(superids, [sc_], _splat_i32(s),
                               mask=hit & lane0)
            return sc_ + jnp.where(hit, one, zeros32)

        sc_ = lax.fori_loop(0, _N // 256, p2s_body, zeros32)
        nsup = sc_[0]

        def p2blk_body(h, hc):
            sv_ = superids[pl.ds((h // 16) * 16, 16)]
            s = _take16(sv_, jnp.full((16,), h % 16, _i32))[0]
            for q in range(4):
                blk = s * 4 + q
                br = blockred[pl.ds(blk * 16, 16)]
                hit = _any_splat(br >= tau)
                plsc.store_scatter(hitids, [hc], _splat_i32(blk),
                                   mask=hit & lane0)
                hc = hc + jnp.where(hit, one, zeros32)
            return hc

        hc = lax.fori_loop(0, nsup, p2blk_body, zeros32)
        nhit = hc[0]

        def p2b_body(h, cc):
            hv = hitids[pl.ds((h // 16) * 16, 16)]
            blk = _take16(hv, jnp.full((16,), h % 16, _i32))[0]
            base = blk * 64
            for k in range(4):
                v = row_v[pl.ds(base + k * 16, 16)]
                msk = v >= tau
                pre, tot = _prefix_total(msk)
                pos = cc + pre - 1
                plsc.store_scatter(candv, [pos], v, mask=msk)
                gi = _splat_i32(base + k * 16) + iota
                plsc.store_scatter(candi, [pos], gi, mask=msk)
                cc = cc + tot
            return cc

        cc = lax.fori_loop(0, nhit, p2b_body, zeros32)
        nc = cc[0]

        for k in range(4):
            pidx = cc + (iota + 16 * k)
            plsc.store_scatter(candv, [pidx], neginf)
            plsc.store_scatter(candi, [pidx], _splat_i32(2 ** 30))

        r0 = _sort64(candv[pl.ds(0, 16)], candv[pl.ds(16, 16)],
                     candv[pl.ds(32, 16)], candv[pl.ds(48, 16)])
        nchunks = (nc + 63) // 64

        def p3_body(c5, r):
            bb = c5 * 64
            cs = _sort64(candv[pl.ds(bb, 16)], candv[pl.ds(bb + 16, 16)],
                         candv[pl.ds(bb + 32, 16)], candv[pl.ds(bb + 48, 16)])
            return _mtop64(r, cs)

        r = lax.fori_loop(1, nchunks, p3_body, r0)
        v64 = _splat_f32(r[3][15])

        nv = (nc + 15) // 16

        def p4gt_body(i, sc):
            v = candv[pl.ds(i * 16, 16)]
            ii = candi[pl.ds(i * 16, 16)]
            msk = v > v64
            pre, tot = _prefix_total(msk)
            pos = sc + pre - 1
            plsc.store_scatter(selv, [pos], v, mask=msk)
            plsc.store_scatter(seli, [pos], ii, mask=msk)
            return sc + tot

        sc = lax.fori_loop(0, nv, p4gt_body, zeros32)

        def p4eq_body(i, sc):
            v = candv[pl.ds(i * 16, 16)]
            ii = candi[pl.ds(i * 16, 16)]
            msk0 = v == v64
            pre, _ = _prefix_total(msk0)
            pos = sc + pre - 1
            msk = msk0 & (pos < _K)
            _, tot = _prefix_total(msk)
            plsc.store_scatter(selv, [pos], v, mask=msk)
            plsc.store_scatter(seli, [pos], ii, mask=msk)
            return sc + tot

        lax.fori_loop(0, nv, p4eq_body, sc)

        sv = tuple(selv[pl.ds(k * 16, 16)] for k in range(4))
        si = tuple(seli[pl.ds(k * 16, 16)] for k in range(4))

        def p5_body(j, ranks):
            jb = (j // 16) * 16
            jl = jnp.full((16,), j % 16, _i32)
            bv = _take16(selv[pl.ds(jb, 16)], jl)
            bi = _take16(seli[pl.ds(jb, 16)], jl)
            out = []
            for k in range(4):
                beats = (bv > sv[k]) | ((bv == sv[k]) & (bi < si[k]))
                out.append(ranks[k] + jnp.where(beats, one, zeros32))
            return tuple(out)

        ranks = lax.fori_loop(0, _K, p5_body, (zeros32,) * 4)
        rsp = _splat_i32(rr)
        for k in range(4):
            plsc.store_scatter(outv, [rsp, ranks[k]], sv[k])
            plsc.store_scatter(outi, [rsp, ranks[k]], si[k])
        return 0

    lax.fori_loop(0, _RPW, do_row, 0)
    pltpu.sync_copy(outv, outv_hbm.at[pl.ds(wid * _RPW, _RPW)])
    pltpu.sync_copy(outi, outi_hbm.at[pl.ds(wid * _RPW, _RPW)])


_sc_topk = pl.kernel(
    _sc_body,
    out_type=[
        jax.ShapeDtypeStruct((_ROWS, _K), _f32),
        jax.ShapeDtypeStruct((_ROWS, _K), _i32),
    ],
    mesh=plsc.VectorSubcoreMesh(core_axis_name="c", subcore_axis_name="s"),
    compiler_params=pltpu.CompilerParams(needs_layout_passes=False),
    scratch_types=[
        pltpu.VMEM((_N,), _f32),
        pltpu.VMEM((_CAP,), _f32),
        pltpu.VMEM((_CAP,), _i32),
        pltpu.VMEM((_N // 64,), _i32),
        pltpu.VMEM((80,), _f32),
        pltpu.VMEM((80,), _i32),
        pltpu.VMEM((_RPW, _K), _f32),
        pltpu.VMEM((_RPW, _K), _i32),
        pltpu.VMEM((_N // 4,), _f32),
        pltpu.VMEM((_N // 16,), _f32),
        pltpu.VMEM((_N // 256,), _i32),
    ],
)


def kernel(accumulated):
    vals, idx = _sc_topk(accumulated)
    return vals, idx.astype(jnp.int64), accumulated


# speedup vs baseline: 9.5365x; 1.0001x over previous
import functools

import jax
import jax.numpy as jnp
from jax import lax
from jax.experimental import pallas as pl
from jax.experimental.pallas import tpu as pltpu
from jax.experimental.pallas import tpu_sc as plsc

_N = 32768
_K = 64
_ROWS = 128
_NW = 32
_RPW = _ROWS // _NW
_CAP = _N + 256

_i32 = jnp.int32
_f32 = jnp.float32


def _vsd(x):
    return lax.rev(lax.sort(x), (0,))


def _rev(x):
    return lax.rev(x, (0,))


def _clean32(x0, x1):
    hi = jnp.maximum(x0, x1)
    lo = jnp.minimum(x0, x1)
    return _vsd(hi), _vsd(lo)


def _merge2(a, b):
    rb = _rev(b)
    return _vsd(jnp.maximum(a, rb)), _vsd(jnp.minimum(a, rb))


def _merge4(a0, a1, b0, b1):
    r0, r1 = _rev(b1), _rev(b0)
    h0, h1 = jnp.maximum(a0, r0), jnp.maximum(a1, r1)
    l0, l1 = jnp.minimum(a0, r0), jnp.minimum(a1, r1)
    ch = _clean32(h0, h1)
    cl = _clean32(l0, l1)
    return ch[0], ch[1], cl[0], cl[1]


def _sort64(v0, v1, v2, v3):
    a = _merge2(_vsd(v0), _vsd(v1))
    b = _merge2(_vsd(v2), _vsd(v3))
    return _merge4(a[0], a[1], b[0], b[1])


def _mtop64(r, c):
    rc = (_rev(c[3]), _rev(c[2]), _rev(c[1]), _rev(c[0]))
    t0 = jnp.maximum(r[0], rc[0])
    t1 = jnp.maximum(r[1], rc[1])
    t2 = jnp.maximum(r[2], rc[2])
    t3 = jnp.maximum(r[3], rc[3])
    p0, p1 = jnp.maximum(t0, t2), jnp.maximum(t1, t3)
    q0, q1 = jnp.minimum(t0, t2), jnp.minimum(t1, t3)
    ch = _clean32(p0, p1)
    cl = _clean32(q0, q1)
    return ch[0], ch[1], cl[0], cl[1]


def _take16(v, idx):
    dnums = lax.GatherDimensionNumbers(
        offset_dims=(), collapsed_slice_dims=(0,), start_index_map=(0,))
    return lax.gather(v, idx[:, None], dnums, (1,),
                      mode=lax.GatherScatterMode.PROMISE_IN_BOUNDS)


def _splat_i32(s):
    return jnp.full((16,), s, _i32)


def _splat_f32(s):
    return jnp.full((16,), s, _f32)


def _prefix_total(m):
    p = plsc.cumsum(jnp.where(m, 1, 0).astype(_i32))
    total = plsc.all_reduce_population_count(m)
    return p, total


def _any_splat(m):
    return plsc.all_reduce_population_count(m) > 0


def _sc_body(acc_hbm, outv_hbm, outi_hbm,
             row_v, candv, candi, hitids, selv, seli, outv, outi,
             blockred, superred, superids):
    wid = lax.axis_index("c") * 16 + lax.axis_index("s")
    iota = lax.iota(_i32, 16)
    lane0 = iota == 0
    neginf = _splat_f32(-jnp.inf)
    zeros32 = _splat_i32(0)
    one = _splat_i32(1)

    def do_row(rr, _):
        row_id = wid * _RPW + rr
        pltpu.sync_copy(acc_hbm.at[row_id], row_v)

        def p1_body(it4, m):
            m = list(m)
            for u in range(4):
                sup = it4 * 4 + u
                base = sup * 256
                sred = None
                for q in range(4):
                    blk = sup * 4 + q
                    v0 = row_v[pl.ds(base + q * 64, 16)]
                    v1 = row_v[pl.ds(base + q * 64 + 16, 16)]
                    v2 = row_v[pl.ds(base + q * 64 + 32, 16)]
                    v3 = row_v[pl.ds(base + q * 64 + 48, 16)]
                    r = jnp.maximum(jnp.maximum(v0, v1), jnp.maximum(v2, v3))
                    blockred[pl.ds(blk * 16, 16)] = r
                    m[u * 4 + q] = jnp.maximum(m[u * 4 + q], r)
                    sred = r if sred is None else jnp.maximum(sred, r)
                superred[pl.ds(sup * 16, 16)] = sred
            return tuple(m)

        m = lax.fori_loop(0, 32, p1_body, (neginf,) * 16)
        a = _sort64(m[0], m[1], m[2], m[3])
        b = _sort64(m[4], m[5], m[6], m[7])
        c4 = _sort64(m[8], m[9], m[10], m[11])
        d = _sort64(m[12], m[13], m[14], m[15])
        t = _mtop64(_mtop64(a, b), _mtop64(c4, d))
        tau = _splat_f32(t[3][15])

        def p2s_body(s, sc_):
            srv = superred[pl.ds(s * 16, 16)]
            hit = _any_splat(srv >= tau)
            plsc.store_scatter(superids, [sc_], _splat_i32(s),
                               mask=hit & lane0)
            return sc_ + jnp.where(hit, one, zeros32)

        sc_ = lax.fori_loop(0, _N // 256, p2s_body, zeros32)
        nsup = sc_[0]

        def p2blk_body(h, hc):
            sv_ = superids[pl.ds((h // 16) * 16, 16)]
            s = _take16(sv_, jnp.full((16,), h % 16, _i32))[0]
            for q in range(4):
                blk = s * 4 + q
                br = blockred[pl.ds(blk * 16, 16)]
                hit = _any_splat(br >= tau)
                plsc.store_scatter(hitids, [hc], _splat_i32(blk),
                                   mask=hit & lane0)
                hc = hc + jnp.where(hit, one, zeros32)
            return hc

        hc = lax.fori_loop(0, nsup, p2blk_body, zeros32)
        nhit = hc[0]

        def p2b_body(h, cc):
            hv = hitids[pl.ds((h // 16) * 16, 16)]
            blk = _take16(hv, jnp.full((16,), h % 16, _i32))[0]
            base = blk * 64
            for k in range(4):
                v = row_v[pl.ds(base + k * 16, 16)]
                msk = v >= tau
                pre, tot = _prefix_total(msk)
                pos = cc + pre - 1
                plsc.store_scatter(candv, [pos], v, mask=msk)
                gi = _splat_i32(base + k * 16) + iota
                plsc.store_scatter(candi, [pos], gi, mask=msk)
                cc = cc + tot
            return cc

        cc = lax.fori_loop(0, nhit, p2b_body, zeros32)
        nc = cc[0]

        for k in range(4):
            pidx = cc + (iota + 16 * k)
            plsc.store_scatter(candv, [pidx], neginf)
            plsc.store_scatter(candi, [pidx], _splat_i32(2 ** 30))

        r0 = _sort64(candv[pl.ds(0, 16)], candv[pl.ds(16, 16)],
                     candv[pl.ds(32, 16)], candv[pl.ds(48, 16)])
        nchunks = (nc + 63) // 64

        def p3_body(c5, r):
            bb = c5 * 64
            cs = _sort64(candv[pl.ds(bb, 16)], candv[pl.ds(bb + 16, 16)],
                         candv[pl.ds(bb + 32, 16)], candv[pl.ds(bb + 48, 16)])
            return _mtop64(r, cs)

        r = lax.fori_loop(1, nchunks, p3_body, r0)
        v64 = _splat_f32(r[3][15])

        nv = (nc + 15) // 16

        def p4gt_body(i, sc):
            v = candv[pl.ds(i * 16, 16)]
            ii = candi[pl.ds(i * 16, 16)]
            msk = v > v64
            pre, tot = _prefix_total(msk)
            pos = sc + pre - 1
            plsc.store_scatter(selv, [pos], v, mask=msk)
            plsc.store_scatter(seli, [pos], ii, mask=msk)
            return sc + tot

        sc = lax.fori_loop(0, nv, p4gt_body, zeros32)

        def p4eq_body(i, sc):
            v = candv[pl.ds(i * 16, 16)]
            ii = candi[pl.ds(i * 16, 16)]
            msk0 = v == v64
            pre, _ = _prefix_total(msk0)
            pos = sc + pre - 1
            msk = msk0 & (pos < _K)
            _, tot = _prefix_total(msk)
            plsc.store_scatter(selv, [pos], v, mask=msk)
            plsc.store_scatter(seli, [pos], ii, mask=msk)
            return sc + tot

        lax.fori_loop(0, nv, p4eq_body, sc)

        sv = tuple(selv[pl.ds(k * 16, 16)] for k in range(4))
        si = tuple(seli[pl.ds(k * 16, 16)] for k in range(4))

        def p5_body(j, ranks):
            jb = (j // 16) * 16
            jl = jnp.full((16,), j % 16, _i32)
            bv = _take16(selv[pl.ds(jb, 16)], jl)
            bi = _take16(seli[pl.ds(jb, 16)], jl)
            out = []
            for k in range(4):
                beats = (bv > sv[k]) | ((bv == sv[k]) & (bi < si[k]))
                out.append(ranks[k] + jnp.where(beats, one, zeros32))
            return tuple(out)

        ranks = lax.fori_loop(0, _K, p5_body, (zeros32,) * 4)
        rsp = _splat_i32(rr)
        for k in range(4):
            plsc.store_scatter(outv, [rsp, ranks[k]], sv[k])
            plsc.store_scatter(outi, [rsp, ranks[k]], si[k])
        return 0

    lax.fori_loop(0, _RPW, do_row, 0)
    pltpu.sync_copy(outv, outv_hbm.at[pl.ds(wid * _RPW, _RPW)])
    pltpu.sync_copy(outi, outi_hbm.at[pl.ds(wid * _RPW, _RPW)])


_sc_topk = pl.kernel(
    _sc_body,
    out_type=[
        jax.ShapeDtypeStruct((_ROWS, _K), _f32),
        jax.ShapeDtypeStruct((_ROWS, _K), _i32),
    ],
    mesh=plsc.VectorSubcoreMesh(core_axis_name="c", subcore_axis_name="s"),
    compiler_params=pltpu.CompilerParams(needs_layout_passes=False),
    scratch_types=[
        pltpu.VMEM((_N,), _f32),
        pltpu.VMEM((_CAP,), _f32),
        pltpu.VMEM((_CAP,), _i32),
        pltpu.VMEM((_N // 64,), _i32),
        pltpu.VMEM((80,), _f32),
        pltpu.VMEM((80,), _i32),
        pltpu.VMEM((_RPW, _K), _f32),
        pltpu.VMEM((_RPW, _K), _i32),
        pltpu.VMEM((_N // 4,), _f32),
        pltpu.VMEM((_N // 16,), _f32),
        pltpu.VMEM((_N // 256,), _i32),
    ],
)


def kernel(accumulated):
    vals, idx = _sc_topk(accumulated)
    return vals, idx.astype(jnp.int64), accumulated


# speedup vs baseline: 11.2264x; 1.1772x over previous
import functools

import jax
import jax.numpy as jnp
from jax import lax
from jax.experimental import pallas as pl
from jax.experimental.pallas import tpu as pltpu
from jax.experimental.pallas import tpu_sc as plsc

_N = 32768
_K = 64
_ROWS = 128
_NW = 32
_RPW = _ROWS // _NW
_CAP = _N + 256

_i32 = jnp.int32
_f32 = jnp.float32


def _vsd(x):
    return lax.rev(lax.sort(x), (0,))


def _rev(x):
    return lax.rev(x, (0,))


def _clean32(x0, x1):
    hi = jnp.maximum(x0, x1)
    lo = jnp.minimum(x0, x1)
    return _vsd(hi), _vsd(lo)


def _merge2(a, b):
    rb = _rev(b)
    return _vsd(jnp.maximum(a, rb)), _vsd(jnp.minimum(a, rb))


def _merge4(a0, a1, b0, b1):
    r0, r1 = _rev(b1), _rev(b0)
    h0, h1 = jnp.maximum(a0, r0), jnp.maximum(a1, r1)
    l0, l1 = jnp.minimum(a0, r0), jnp.minimum(a1, r1)
    ch = _clean32(h0, h1)
    cl = _clean32(l0, l1)
    return ch[0], ch[1], cl[0], cl[1]


def _sort64(v0, v1, v2, v3):
    a = _merge2(_vsd(v0), _vsd(v1))
    b = _merge2(_vsd(v2), _vsd(v3))
    return _merge4(a[0], a[1], b[0], b[1])


def _mtop64(r, c):
    rc = (_rev(c[3]), _rev(c[2]), _rev(c[1]), _rev(c[0]))
    t0 = jnp.maximum(r[0], rc[0])
    t1 = jnp.maximum(r[1], rc[1])
    t2 = jnp.maximum(r[2], rc[2])
    t3 = jnp.maximum(r[3], rc[3])
    p0, p1 = jnp.maximum(t0, t2), jnp.maximum(t1, t3)
    q0, q1 = jnp.minimum(t0, t2), jnp.minimum(t1, t3)
    ch = _clean32(p0, p1)
    cl = _clean32(q0, q1)
    return ch[0], ch[1], cl[0], cl[1]


def _take16(v, idx):
    dnums = lax.GatherDimensionNumbers(
        offset_dims=(), collapsed_slice_dims=(0,), start_index_map=(0,))
    return lax.gather(v, idx[:, None], dnums, (1,),
                      mode=lax.GatherScatterMode.PROMISE_IN_BOUNDS)


def _splat_i32(s):
    return jnp.full((16,), s, _i32)


def _splat_f32(s):
    return jnp.full((16,), s, _f32)


def _prefix_total(m):
    p = plsc.cumsum(jnp.where(m, 1, 0).astype(_i32))
    total = plsc.all_reduce_population_count(m)
    return p, total


def _any_splat(m):
    return plsc.all_reduce_population_count(m) > 0


def _sc_body(acc_hbm, bmax_hbm, outv_hbm, outi_hbm,
             row_v, candv, candi, hitids, selv, seli, outv, outi, bm_v):
    wid = lax.axis_index("c") * 16 + lax.axis_index("s")
    iota = lax.iota(_i32, 16)
    lane0 = iota == 0
    neginf = _splat_f32(-jnp.inf)
    zeros32 = _splat_i32(0)
    one = _splat_i32(1)

    def do_row(rr, _):
        row_id = wid * _RPW + rr
        pltpu.sync_copy(acc_hbm.at[row_id], row_v)
        pltpu.sync_copy(bmax_hbm.at[row_id], bm_v)

        def bmv(i):
            return bm_v[pl.ds(i * 16, 16)]

        t = _sort64(bmv(0), bmv(1), bmv(2), bmv(3))
        for g in range(1, 8):
            cs = _sort64(bmv(4 * g), bmv(4 * g + 1),
                         bmv(4 * g + 2), bmv(4 * g + 3))
            t = _mtop64(t, cs)
        tau = _splat_f32(t[3][15])

        def p2a_body(i, hc):
            msk = bmv(i) >= tau
            pre, tot = _prefix_total(msk)
            plsc.store_scatter(hitids, [hc + pre - 1],
                               _splat_i32(i * 16) + iota, mask=msk)
            return hc + tot

        hc = lax.fori_loop(0, _N // 64 // 16, p2a_body, zeros32)
        nhit = hc[0]

        def p2b_body(h, cc):
            hv = hitids[pl.ds((h // 16) * 16, 16)]
            blk = _take16(hv, jnp.full((16,), h % 16, _i32))[0]
            base = blk * 64
            for k in range(4):
                v = row_v[pl.ds(base + k * 16, 16)]
                msk = v >= tau
                pre, tot = _prefix_total(msk)
                pos = cc + pre - 1
                plsc.store_scatter(candv, [pos], v, mask=msk)
                gi = _splat_i32(base + k * 16) + iota
                plsc.store_scatter(candi, [pos], gi, mask=msk)
                cc = cc + tot
            return cc

        cc = lax.fori_loop(0, nhit, p2b_body, zeros32)
        nc = cc[0]

        for k in range(4):
            pidx = cc + (iota + 16 * k)
            plsc.store_scatter(candv, [pidx], neginf)
            plsc.store_scatter(candi, [pidx], _splat_i32(2 ** 30))

        r0 = _sort64(candv[pl.ds(0, 16)], candv[pl.ds(16, 16)],
                     candv[pl.ds(32, 16)], candv[pl.ds(48, 16)])
        nchunks = (nc + 63) // 64

        def p3_body(c5, r):
            bb = c5 * 64
            cs = _sort64(candv[pl.ds(bb, 16)], candv[pl.ds(bb + 16, 16)],
                         candv[pl.ds(bb + 32, 16)], candv[pl.ds(bb + 48, 16)])
            return _mtop64(r, cs)

        r = lax.fori_loop(1, nchunks, p3_body, r0)
        v64 = _splat_f32(r[3][15])

        nv = (nc + 15) // 16

        def p4gt_body(i, sc):
            v = candv[pl.ds(i * 16, 16)]
            ii = candi[pl.ds(i * 16, 16)]
            msk = v > v64
            pre, tot = _prefix_total(msk)
            pos = sc + pre - 1
            plsc.store_scatter(selv, [pos], v, mask=msk)
            plsc.store_scatter(seli, [pos], ii, mask=msk)
            return sc + tot

        sc = lax.fori_loop(0, nv, p4gt_body, zeros32)

        def p4eq_body(i, sc):
            v = candv[pl.ds(i * 16, 16)]
            ii = candi[pl.ds(i * 16, 16)]
            msk0 = v == v64
            pre, _ = _prefix_total(msk0)
            pos = sc + pre - 1
            msk = msk0 & (pos < _K)
            _, tot = _prefix_total(msk)
            plsc.store_scatter(selv, [pos], v, mask=msk)
            plsc.store_scatter(seli, [pos], ii, mask=msk)
            return sc + tot

        lax.fori_loop(0, nv, p4eq_body, sc)

        sv = tuple(selv[pl.ds(k * 16, 16)] for k in range(4))
        si = tuple(seli[pl.ds(k * 16, 16)] for k in range(4))

        def p5_body(j, ranks):
            jb = (j // 16) * 16
            jl = jnp.full((16,), j % 16, _i32)
            bv = _take16(selv[pl.ds(jb, 16)], jl)
            bi = _take16(seli[pl.ds(jb, 16)], jl)
            out = []
            for k in range(4):
                beats = (bv > sv[k]) | ((bv == sv[k]) & (bi < si[k]))
                out.append(ranks[k] + jnp.where(beats, one, zeros32))
            return tuple(out)

        ranks = lax.fori_loop(0, _K, p5_body, (zeros32,) * 4)
        rsp = _splat_i32(rr)
        for k in range(4):
            plsc.store_scatter(outv, [rsp, ranks[k]], sv[k])
            plsc.store_scatter(outi, [rsp, ranks[k]], si[k])
        return 0

    lax.fori_loop(0, _RPW, do_row, 0)
    pltpu.sync_copy(outv, outv_hbm.at[pl.ds(wid * _RPW, _RPW)])
    pltpu.sync_copy(outi, outi_hbm.at[pl.ds(wid * _RPW, _RPW)])


_sc_topk = pl.kernel(
    _sc_body,
    out_type=[
        jax.ShapeDtypeStruct((_ROWS, _K), _f32),
        jax.ShapeDtypeStruct((_ROWS, _K), _i32),
    ],
    mesh=plsc.VectorSubcoreMesh(core_axis_name="c", subcore_axis_name="s"),
    compiler_params=pltpu.CompilerParams(needs_layout_passes=False),
    scratch_types=[
        pltpu.VMEM((_N,), _f32),
        pltpu.VMEM((_CAP,), _f32),
        pltpu.VMEM((_CAP,), _i32),
        pltpu.VMEM((_N // 64,), _i32),
        pltpu.VMEM((80,), _f32),
        pltpu.VMEM((80,), _i32),
        pltpu.VMEM((_RPW, _K), _f32),
        pltpu.VMEM((_RPW, _K), _i32),
        pltpu.VMEM((_N // 64,), _f32),
    ],
)


def _tc_body(x_ref, o_ref):
    x = x_ref[...]
    o_ref[...] = jnp.max(x.reshape(x.shape[0], _N // 64, 64), axis=-1)


_tc_blockmax = pl.pallas_call(
    _tc_body,
    grid=(8,),
    in_specs=[pl.BlockSpec((_ROWS // 8, _N), lambda i: (i, 0))],
    out_specs=pl.BlockSpec((_ROWS // 8, _N // 64), lambda i: (i, 0)),
    out_shape=jax.ShapeDtypeStruct((_ROWS, _N // 64), _f32),
)


def kernel(accumulated):
    vals, idx = _sc_topk(accumulated, _tc_blockmax(accumulated))
    return vals, idx.astype(jnp.int64), accumulated


# speedup vs baseline: 12.9324x; 1.1520x over previous
import functools

import jax
import jax.numpy as jnp
from jax import lax
from jax.experimental import pallas as pl
from jax.experimental.pallas import tpu as pltpu
from jax.experimental.pallas import tpu_sc as plsc

_N = 32768
_K = 64
_ROWS = 128
_NW = 32
_RPW = _ROWS // _NW
_CAP = _N + 256

_i32 = jnp.int32
_f32 = jnp.float32


def _vsd(x):
    return lax.rev(lax.sort(x), (0,))


def _rev(x):
    return lax.rev(x, (0,))


def _clean32(x0, x1):
    hi = jnp.maximum(x0, x1)
    lo = jnp.minimum(x0, x1)
    return _vsd(hi), _vsd(lo)


def _merge2(a, b):
    rb = _rev(b)
    return _vsd(jnp.maximum(a, rb)), _vsd(jnp.minimum(a, rb))


def _merge4(a0, a1, b0, b1):
    r0, r1 = _rev(b1), _rev(b0)
    h0, h1 = jnp.maximum(a0, r0), jnp.maximum(a1, r1)
    l0, l1 = jnp.minimum(a0, r0), jnp.minimum(a1, r1)
    ch = _clean32(h0, h1)
    cl = _clean32(l0, l1)
    return ch[0], ch[1], cl[0], cl[1]


def _sort64(v0, v1, v2, v3):
    a = _merge2(_vsd(v0), _vsd(v1))
    b = _merge2(_vsd(v2), _vsd(v3))
    return _merge4(a[0], a[1], b[0], b[1])


def _mtop64(r, c):
    rc = (_rev(c[3]), _rev(c[2]), _rev(c[1]), _rev(c[0]))
    t0 = jnp.maximum(r[0], rc[0])
    t1 = jnp.maximum(r[1], rc[1])
    t2 = jnp.maximum(r[2], rc[2])
    t3 = jnp.maximum(r[3], rc[3])
    p0, p1 = jnp.maximum(t0, t2), jnp.maximum(t1, t3)
    q0, q1 = jnp.minimum(t0, t2), jnp.minimum(t1, t3)
    ch = _clean32(p0, p1)
    cl = _clean32(q0, q1)
    return ch[0], ch[1], cl[0], cl[1]


def _take16(v, idx):
    dnums = lax.GatherDimensionNumbers(
        offset_dims=(), collapsed_slice_dims=(0,), start_index_map=(0,))
    return lax.gather(v, idx[:, None], dnums, (1,),
                      mode=lax.GatherScatterMode.PROMISE_IN_BOUNDS)


def _splat_i32(s):
    return jnp.full((16,), s, _i32)


def _splat_f32(s):
    return jnp.full((16,), s, _f32)


def _prefix_total(m):
    p = plsc.cumsum(jnp.where(m, 1, 0).astype(_i32))
    total = plsc.all_reduce_population_count(m)
    return p, total


def _any_splat(m):
    return plsc.all_reduce_population_count(m) > 0


def _sc_body(acc_hbm, bmax_hbm, outv_hbm, outi_hbm,
             row_v, candv, candi, hitids, selv, seli, outv, outi, bm_v):
    wid = lax.axis_index("c") * 16 + lax.axis_index("s")
    iota = lax.iota(_i32, 16)
    lane0 = iota == 0
    neginf = _splat_f32(-jnp.inf)
    zeros32 = _splat_i32(0)
    one = _splat_i32(1)

    def do_row(rr, _):
        row_id = wid * _RPW + rr
        pltpu.sync_copy(acc_hbm.at[row_id], row_v)
        pltpu.sync_copy(bmax_hbm.at[row_id], bm_v)

        def bmv(i):
            return bm_v[pl.ds(i * 16, 16)]

        t = _sort64(bmv(0), bmv(1), bmv(2), bmv(3))
        for g in range(1, 8):
            cs = _sort64(bmv(4 * g), bmv(4 * g + 1),
                         bmv(4 * g + 2), bmv(4 * g + 3))
            t = _mtop64(t, cs)
        tau = _splat_f32(t[3][15])

        def p2a_body(i, hc):
            msk = bmv(i) >= tau
            pre, tot = _prefix_total(msk)
            plsc.store_scatter(hitids, [hc + pre - 1],
                               _splat_i32(i * 16) + iota, mask=msk)
            return hc + tot

        hc = lax.fori_loop(0, _N // 64 // 16, p2a_body, zeros32)
        nhit = hc[0]

        def p2b_body(h, cc):
            hv = hitids[pl.ds((h // 16) * 16, 16)]
            blk = _take16(hv, jnp.full((16,), h % 16, _i32))[0]
            base = blk * 64
            for k in range(4):
                v = row_v[pl.ds(base + k * 16, 16)]
                msk = v >= tau
                pre, tot = _prefix_total(msk)
                pos = cc + pre - 1
                plsc.store_scatter(candv, [pos], v, mask=msk)
                gi = _splat_i32(base + k * 16) + iota
                plsc.store_scatter(candi, [pos], gi, mask=msk)
                cc = cc + tot
            return cc

        cc = lax.fori_loop(0, nhit, p2b_body, zeros32)
        nc = cc[0]

        for k in range(4):
            pidx = cc + (iota + 16 * k)
            plsc.store_scatter(candv, [pidx], neginf)
            plsc.store_scatter(candi, [pidx], _splat_i32(2 ** 30))

        r0 = _sort64(candv[pl.ds(0, 16)], candv[pl.ds(16, 16)],
                     candv[pl.ds(32, 16)], candv[pl.ds(48, 16)])
        nchunks = (nc + 63) // 64

        def p3_body(c5, r):
            bb = c5 * 64
            cs = _sort64(candv[pl.ds(bb, 16)], candv[pl.ds(bb + 16, 16)],
                         candv[pl.ds(bb + 32, 16)], candv[pl.ds(bb + 48, 16)])
            return _mtop64(r, cs)

        r = lax.fori_loop(1, nchunks, p3_body, r0)
        v64 = _splat_f32(r[3][15])

        nv = (nc + 15) // 16

        def p4gt_body(i, sc):
            v = candv[pl.ds(i * 16, 16)]
            ii = candi[pl.ds(i * 16, 16)]
            msk = v > v64
            pre, tot = _prefix_total(msk)
            pos = sc + pre - 1
            plsc.store_scatter(selv, [pos], v, mask=msk)
            plsc.store_scatter(seli, [pos], ii, mask=msk)
            return sc + tot

        sc = lax.fori_loop(0, nv, p4gt_body, zeros32)

        def p4eq_body(i, sc):
            v = candv[pl.ds(i * 16, 16)]
            ii = candi[pl.ds(i * 16, 16)]
            msk0 = v == v64
            pre, _ = _prefix_total(msk0)
            pos = sc + pre - 1
            msk = msk0 & (pos < _K)
            _, tot = _prefix_total(msk)
            plsc.store_scatter(selv, [pos], v, mask=msk)
            plsc.store_scatter(seli, [pos], ii, mask=msk)
            return sc + tot

        lax.fori_loop(0, nv, p4eq_body, sc)

        sv = tuple(selv[pl.ds(k * 16, 16)] for k in range(4))
        si = tuple(seli[pl.ds(k * 16, 16)] for k in range(4))

        def p5_body(j, ranks):
            jb = (j // 16) * 16
            jl = jnp.full((16,), j % 16, _i32)
            bv = _take16(selv[pl.ds(jb, 16)], jl)
            bi = _take16(seli[pl.ds(jb, 16)], jl)
            out = []
            for k in range(4):
                beats = (bv > sv[k]) | ((bv == sv[k]) & (bi < si[k]))
                out.append(ranks[k] + jnp.where(beats, one, zeros32))
            return tuple(out)

        ranks = lax.fori_loop(0, _K, p5_body, (zeros32,) * 4)
        rsp = _splat_i32(rr)
        for k in range(4):
            plsc.store_scatter(outv, [rsp, ranks[k]], sv[k])
            plsc.store_scatter(outi, [rsp, ranks[k]], si[k])
        return 0

    lax.fori_loop(0, _RPW, do_row, 0)
    pltpu.sync_copy(outv, outv_hbm.at[pl.ds(wid * _RPW, _RPW)])
    pltpu.sync_copy(outi, outi_hbm.at[pl.ds(wid * _RPW, _RPW)])


_sc_topk = pl.kernel(
    _sc_body,
    out_type=[
        jax.ShapeDtypeStruct((_ROWS, _K), _f32),
        jax.ShapeDtypeStruct((_ROWS, _K), _i32),
    ],
    mesh=plsc.VectorSubcoreMesh(core_axis_name="c", subcore_axis_name="s"),
    compiler_params=pltpu.CompilerParams(needs_layout_passes=False),
    scratch_types=[
        pltpu.VMEM((_N,), _f32),
        pltpu.VMEM((_CAP,), _f32),
        pltpu.VMEM((_CAP,), _i32),
        pltpu.VMEM((_N // 64,), _i32),
        pltpu.VMEM((80,), _f32),
        pltpu.VMEM((80,), _i32),
        pltpu.VMEM((_RPW, _K), _f32),
        pltpu.VMEM((_RPW, _K), _i32),
        pltpu.VMEM((_N // 64,), _f32),
    ],
)


def _tc_body(x_ref, o_ref, p_ref):
    x = x_ref[...]
    o_ref[...] = jnp.max(x.reshape(x.shape[0], _N // 64, 64), axis=-1)
    p_ref[...] = x


_tc_blockmax = pl.pallas_call(
    _tc_body,
    grid=(8,),
    in_specs=[pl.BlockSpec((_ROWS // 8, _N), lambda i: (i, 0))],
    out_specs=[
        pl.BlockSpec((_ROWS // 8, _N // 64), lambda i: (i, 0)),
        pl.BlockSpec((_ROWS // 8, _N), lambda i: (i, 0)),
    ],
    out_shape=[
        jax.ShapeDtypeStruct((_ROWS, _N // 64), _f32),
        jax.ShapeDtypeStruct((_ROWS, _N), _f32),
    ],
)


def kernel(accumulated):
    bmax, passthrough = _tc_blockmax(accumulated)
    vals, idx = _sc_topk(accumulated, bmax)
    return vals, idx.astype(jnp.int64), passthrough
